# baseline (speedup 1.0000x reference)
import functools
import math

import jax
import jax.numpy as jnp
from jax.experimental import pallas as pl
from jax.experimental.pallas import tpu as pltpu

H = 16
W_BLK = 128
TOK = 256
NSUB = TOK // W_BLK
CVT_ROWS = 256

_TRANS = (((1,), (1,)), ((), ()))


def _fused_attn_kernel(x0_ref, x_ref, wq_hbm, wk_hbm, wv_hbm, wo_hbm, out_ref,
                       wq_s, wk_s, wv_s, wo_s, stg,
                       qa, ka, va, qb, kb, vb,
                       s_scr, o_scr, sems, *, inv_scale, d, nsteps):
    i = pl.program_id(0)
    nch = d // CVT_ROWS
    srcs = (wq_hbm, wk_hbm, wv_hbm, wo_hbm)
    dsts = (wq_s, wk_s, wv_s, wo_s)
    ntot = 4 * nch

    def project(xv, qw, kw, vw):
        qw[...] = jax.lax.dot_general(
            xv, wq_s[...], _TRANS,
            preferred_element_type=jnp.float32).astype(jnp.bfloat16)
        kw[...] = jax.lax.dot_general(
            xv, wk_s[...], _TRANS,
            preferred_element_type=jnp.float32).astype(jnp.bfloat16)
        vw[...] = jax.lax.dot_general(
            xv, wv_s[...], _TRANS,
            preferred_element_type=jnp.float32).astype(jnp.bfloat16)

    def attn_out(qr, kr, vr):
        for h in range(H):
            cs = slice(h * W_BLK, (h + 1) * W_BLK)
            for j in range(NSUB):
                rs = slice(j * W_BLK, (j + 1) * W_BLK)
                b = h * NSUB + j
                s_scr[b * W_BLK:(b + 1) * W_BLK, :] = jax.lax.dot_general(
                    qr[rs, cs], kr[rs, cs], _TRANS,
                    preferred_element_type=jnp.float32)

        sv = s_scr[...]
        sv = (sv - jnp.max(sv, axis=-1, keepdims=True)) * inv_scale
        p = jnp.exp(sv)
        p = (p / jnp.sum(p, axis=-1, keepdims=True)).astype(jnp.bfloat16)

        for h in range(H):
            cs = slice(h * W_BLK, (h + 1) * W_BLK)
            for j in range(NSUB):
                rs = slice(j * W_BLK, (j + 1) * W_BLK)
                b = h * NSUB + j
                o_scr[rs, cs] = jnp.dot(
                    p[b * W_BLK:(b + 1) * W_BLK, :], vr[rs, cs],
                    preferred_element_type=jnp.float32).astype(jnp.bfloat16)

        out_ref[...] = jax.lax.dot_general(
            o_scr[...], wo_s[...], _TRANS,
            preferred_element_type=jnp.float32)

    @pl.when(i == 0)
    def _convert_and_prime():
        def dma(t, buf):
            w, c = divmod(t, nch)
            return pltpu.make_async_copy(
                srcs[w].at[pl.ds(c * CVT_ROWS, CVT_ROWS), :],
                stg.at[buf], sems.at[buf])

        dma(0, 0).start()
        for t in range(ntot):
            buf = t % 2
            if t + 1 < ntot:
                dma(t + 1, 1 - buf).start()
            dma(t, buf).wait()
            w, c = divmod(t, nch)
            dsts[w][c * CVT_ROWS:(c + 1) * CVT_ROWS, :] = (
                stg[buf].astype(jnp.bfloat16))
        project(x0_ref[...].astype(jnp.bfloat16), qa, ka, va)

    even = i % 2 == 0
    more = i + 1 < nsteps

    @pl.when(even & more)
    def _():
        project(x_ref[...].astype(jnp.bfloat16), qb, kb, vb)

    @pl.when((~even) & more)
    def _():
        project(x_ref[...].astype(jnp.bfloat16), qa, ka, va)

    @pl.when(even)
    def _():
        attn_out(qa, ka, va)

    @pl.when(~even)
    def _():
        attn_out(qb, kb, vb)


def kernel(x, Wq, Wk, Wv, Wo):
    B_, T_, D_ = x.shape
    N = B_ * T_
    Dh = D_ // H
    inv_scale = 1.0 / math.sqrt(Dh)
    nsteps = N // TOK

    x2 = x.reshape(N, D_)
    body = functools.partial(_fused_attn_kernel, inv_scale=inv_scale, d=D_,
                             nsteps=nsteps)
    bf = jnp.bfloat16
    out = pl.pallas_call(
        body,
        grid=(nsteps,),
        in_specs=[
            pl.BlockSpec((TOK, D_), lambda i: (0, 0)),
            pl.BlockSpec((TOK, D_),
                         lambda i: (jnp.minimum(i + 1, nsteps - 1), 0)),
            pl.BlockSpec(memory_space=pl.ANY),
            pl.BlockSpec(memory_space=pl.ANY),
            pl.BlockSpec(memory_space=pl.ANY),
            pl.BlockSpec(memory_space=pl.ANY),
        ],
        out_specs=pl.BlockSpec((TOK, D_), lambda i: (i, 0)),
        out_shape=jax.ShapeDtypeStruct((N, D_), jnp.float32),
        scratch_shapes=[
            pltpu.VMEM((D_, D_), bf),
            pltpu.VMEM((D_, D_), bf),
            pltpu.VMEM((D_, D_), bf),
            pltpu.VMEM((D_, D_), bf),
            pltpu.VMEM((2, CVT_ROWS, D_), jnp.float32),
            pltpu.VMEM((TOK, D_), bf),
            pltpu.VMEM((TOK, D_), bf),
            pltpu.VMEM((TOK, D_), bf),
            pltpu.VMEM((TOK, D_), bf),
            pltpu.VMEM((TOK, D_), bf),
            pltpu.VMEM((TOK, D_), bf),
            pltpu.VMEM((H * NSUB * W_BLK, W_BLK), jnp.float32),
            pltpu.VMEM((TOK, D_), bf),
            pltpu.SemaphoreType.DMA((2,)),
        ],
        compiler_params=pltpu.CompilerParams(
            dimension_semantics=("arbitrary",),
        ),
    )(x2, x2, Wq, Wk, Wv, Wo)
    return out.reshape(B_, T_, D_)


# speedup vs baseline: 1.0057x; 1.0057x over previous
import functools
import math

import jax
import jax.numpy as jnp
from jax.experimental import pallas as pl
from jax.experimental.pallas import tpu as pltpu

H = 16
W_BLK = 128
TOK = 256
NSUB = TOK // W_BLK
CVT_ROWS = 256

_TRANS = (((1,), (1,)), ((), ()))


def _fused_attn_kernel(x0_ref, x_ref, wq_hbm, wk_hbm, wv_hbm, wo_hbm, out_ref,
                       wq_s, wk_s, wv_s, wo_s, stg,
                       qa, ka, va, qb, kb, vb,
                       s_scr, o_scr, sems, *, inv_scale, d, nsteps):
    i = pl.program_id(0)
    nch = d // CVT_ROWS
    srcs = (wq_hbm, wk_hbm, wv_hbm, wo_hbm)
    dsts = (wq_s, wk_s, wv_s, wo_s)
    ntot = 4 * nch

    def project(xv, qw, kw, vw):
        qw[...] = jax.lax.dot_general(
            xv, wq_s[...], _TRANS,
            preferred_element_type=jnp.float32).astype(jnp.bfloat16)
        kw[...] = jax.lax.dot_general(
            xv, wk_s[...], _TRANS,
            preferred_element_type=jnp.float32).astype(jnp.bfloat16)
        vw[...] = jax.lax.dot_general(
            xv, wv_s[...], _TRANS,
            preferred_element_type=jnp.float32).astype(jnp.bfloat16)

    def attn_out(qr, kr, vr):
        for h in range(H):
            cs = slice(h * W_BLK, (h + 1) * W_BLK)
            for j in range(NSUB):
                rs = slice(j * W_BLK, (j + 1) * W_BLK)
                b = h * NSUB + j
                s_scr[b * W_BLK:(b + 1) * W_BLK, :] = jax.lax.dot_general(
                    qr[rs, cs], kr[rs, cs], _TRANS,
                    preferred_element_type=jnp.float32)

        sv = s_scr[...]
        sv = (sv - jnp.max(sv, axis=-1, keepdims=True)) * inv_scale
        p = jnp.exp(sv)
        p = (p / jnp.sum(p, axis=-1, keepdims=True)).astype(jnp.bfloat16)

        for h in range(H):
            cs = slice(h * W_BLK, (h + 1) * W_BLK)
            for j in range(NSUB):
                rs = slice(j * W_BLK, (j + 1) * W_BLK)
                b = h * NSUB + j
                o_scr[rs, cs] = jnp.dot(
                    p[b * W_BLK:(b + 1) * W_BLK, :], vr[rs, cs],
                    preferred_element_type=jnp.float32).astype(jnp.bfloat16)

        out_ref[...] = jax.lax.dot_general(
            o_scr[...], wo_s[...], _TRANS,
            preferred_element_type=jnp.float32)

    @pl.when(i == 0)
    def _convert_and_prime():
        def dma(t, buf):
            w, c = divmod(t, nch)
            return pltpu.make_async_copy(
                srcs[w].at[pl.ds(c * CVT_ROWS, CVT_ROWS), :],
                stg.at[buf], sems.at[buf])

        dma(0, 0).start()
        for t in range(ntot):
            buf = t % 2
            if t + 1 < ntot:
                dma(t + 1, 1 - buf).start()
            dma(t, buf).wait()
            w, c = divmod(t, nch)
            dsts[w][c * CVT_ROWS:(c + 1) * CVT_ROWS, :] = (
                stg[buf].astype(jnp.bfloat16))
        project(x0_ref[...].astype(jnp.bfloat16), qa, ka, va)

    even = i % 2 == 0
    more = i + 1 < nsteps

    @pl.when(even)
    def _():
        project(x_ref[...].astype(jnp.bfloat16), qb, kb, vb)
        attn_out(qa, ka, va)

    @pl.when((~even) & more)
    def _():
        project(x_ref[...].astype(jnp.bfloat16), qa, ka, va)
        attn_out(qb, kb, vb)

    @pl.when((~even) & (~more))
    def _():
        attn_out(qb, kb, vb)


def kernel(x, Wq, Wk, Wv, Wo):
    B_, T_, D_ = x.shape
    N = B_ * T_
    Dh = D_ // H
    inv_scale = 1.0 / math.sqrt(Dh)
    nsteps = N // TOK

    x2 = x.reshape(N, D_)
    body = functools.partial(_fused_attn_kernel, inv_scale=inv_scale, d=D_,
                             nsteps=nsteps)
    bf = jnp.bfloat16
    out = pl.pallas_call(
        body,
        grid=(nsteps,),
        in_specs=[
            pl.BlockSpec((TOK, D_), lambda i: (0, 0)),
            pl.BlockSpec((TOK, D_),
                         lambda i: (jnp.minimum(i + 1, nsteps - 1), 0)),
            pl.BlockSpec(memory_space=pl.ANY),
            pl.BlockSpec(memory_space=pl.ANY),
            pl.BlockSpec(memory_space=pl.ANY),
            pl.BlockSpec(memory_space=pl.ANY),
        ],
        out_specs=pl.BlockSpec((TOK, D_), lambda i: (i, 0)),
        out_shape=jax.ShapeDtypeStruct((N, D_), jnp.float32),
        scratch_shapes=[
            pltpu.VMEM((D_, D_), bf),
            pltpu.VMEM((D_, D_), bf),
            pltpu.VMEM((D_, D_), bf),
            pltpu.VMEM((D_, D_), bf),
            pltpu.VMEM((2, CVT_ROWS, D_), jnp.float32),
            pltpu.VMEM((TOK, D_), bf),
            pltpu.VMEM((TOK, D_), bf),
            pltpu.VMEM((TOK, D_), bf),
            pltpu.VMEM((TOK, D_), bf),
            pltpu.VMEM((TOK, D_), bf),
            pltpu.VMEM((TOK, D_), bf),
            pltpu.VMEM((H * NSUB * W_BLK, W_BLK), jnp.float32),
            pltpu.VMEM((TOK, D_), bf),
            pltpu.SemaphoreType.DMA((2,)),
        ],
        compiler_params=pltpu.CompilerParams(
            dimension_semantics=("arbitrary",),
        ),
    )(x2, x2, Wq, Wk, Wv, Wo)
    return out.reshape(B_, T_, D_)


# speedup vs baseline: 1.1025x; 1.0962x over previous
import functools
import math

import jax
import jax.numpy as jnp
from jax.experimental import pallas as pl
from jax.experimental.pallas import tpu as pltpu

H = 16
W_BLK = 128
TOK = 256
NSUB = TOK // W_BLK
CVT_ROWS = 512

_TRANS = (((1,), (1,)), ((), ()))


def _fused_attn_kernel(x_ref, wq_hbm, wk_hbm, wv_hbm, wo_hbm, out_ref,
                       wq_s, wk_s, wv_s, wo_s, stg, s_scr, o_scr, sems,
                       *, inv_scale, d):
    i = pl.program_id(0)
    nch = d // CVT_ROWS
    srcs = (wq_hbm, wk_hbm, wv_hbm, wo_hbm)
    dsts = (wq_s, wk_s, wv_s, wo_s)
    ntot = 4 * nch

    def dma(t, buf):
        w, c = divmod(t, nch)
        return pltpu.make_async_copy(
            srcs[w].at[pl.ds(c * CVT_ROWS, CVT_ROWS), :],
            stg.at[buf], sems.at[buf])

    def proj(xv, w_s):
        return jax.lax.dot_general(xv, w_s[...], _TRANS,
                                   preferred_element_type=jnp.float32)

    def attn(q, k, v):
        for h in range(H):
            cs = slice(h * W_BLK, (h + 1) * W_BLK)
            qh = q[:, cs]
            kh = k[:, cs]
            for j in range(NSUB):
                rs = slice(j * W_BLK, (j + 1) * W_BLK)
                b = h * NSUB + j
                s_scr[b * W_BLK:(b + 1) * W_BLK, :] = jax.lax.dot_general(
                    qh[rs, :], kh[rs, :], _TRANS,
                    preferred_element_type=jnp.float32)

        sv = s_scr[...]
        sv = (sv - jnp.max(sv, axis=-1, keepdims=True)) * inv_scale
        p = jnp.exp(sv)
        p = p / jnp.sum(p, axis=-1, keepdims=True)

        for h in range(H):
            cs = slice(h * W_BLK, (h + 1) * W_BLK)
            vh = v[:, cs]
            for j in range(NSUB):
                rs = slice(j * W_BLK, (j + 1) * W_BLK)
                b = h * NSUB + j
                o_scr[rs, cs] = jnp.dot(
                    p[b * W_BLK:(b + 1) * W_BLK, :], vh[rs, :],
                    preferred_element_type=jnp.float32).astype(jnp.bfloat16)

    @pl.when(i == 0)
    def _convert_and_compute():
        state = [0]
        dma(0, 0).start()

        def cvt_next_weight():
            for _ in range(nch):
                t = state[0]
                buf = t % 2
                if t + 1 < ntot:
                    dma(t + 1, 1 - buf).start()
                dma(t, buf).wait()
                w, c = divmod(t, nch)
                dsts[w][c * CVT_ROWS:(c + 1) * CVT_ROWS, :] = (
                    stg[buf].astype(jnp.bfloat16))
                state[0] = t + 1

        xv = x_ref[...].astype(jnp.bfloat16)
        cvt_next_weight()
        q = proj(xv, wq_s)
        cvt_next_weight()
        k = proj(xv, wk_s)
        cvt_next_weight()
        v = proj(xv, wv_s)
        attn(q, k, v)
        cvt_next_weight()
        out_ref[...] = jax.lax.dot_general(
            o_scr[...], wo_s[...], _TRANS,
            preferred_element_type=jnp.float32)

    @pl.when(i != 0)
    def _compute():
        xv = x_ref[...].astype(jnp.bfloat16)
        q = proj(xv, wq_s)
        k = proj(xv, wk_s)
        v = proj(xv, wv_s)
        attn(q, k, v)
        out_ref[...] = jax.lax.dot_general(
            o_scr[...], wo_s[...], _TRANS,
            preferred_element_type=jnp.float32)


def kernel(x, Wq, Wk, Wv, Wo):
    B_, T_, D_ = x.shape
    N = B_ * T_
    Dh = D_ // H
    inv_scale = 1.0 / math.sqrt(Dh)

    x2 = x.reshape(N, D_)
    body = functools.partial(_fused_attn_kernel, inv_scale=inv_scale, d=D_)
    out = pl.pallas_call(
        body,
        grid=(N // TOK,),
        in_specs=[
            pl.BlockSpec((TOK, D_), lambda i: (i, 0)),
            pl.BlockSpec(memory_space=pl.ANY),
            pl.BlockSpec(memory_space=pl.ANY),
            pl.BlockSpec(memory_space=pl.ANY),
            pl.BlockSpec(memory_space=pl.ANY),
        ],
        out_specs=pl.BlockSpec((TOK, D_), lambda i: (i, 0)),
        out_shape=jax.ShapeDtypeStruct((N, D_), jnp.float32),
        scratch_shapes=[
            pltpu.VMEM((D_, D_), jnp.bfloat16),
            pltpu.VMEM((D_, D_), jnp.bfloat16),
            pltpu.VMEM((D_, D_), jnp.bfloat16),
            pltpu.VMEM((D_, D_), jnp.bfloat16),
            pltpu.VMEM((2, CVT_ROWS, D_), jnp.float32),
            pltpu.VMEM((H * NSUB * W_BLK, W_BLK), jnp.float32),
            pltpu.VMEM((TOK, D_), jnp.bfloat16),
            pltpu.SemaphoreType.DMA((2,)),
        ],
        compiler_params=pltpu.CompilerParams(
            dimension_semantics=("arbitrary",),
        ),
    )(x2, Wq, Wk, Wv, Wo)
    return out.reshape(B_, T_, D_)


# speedup vs baseline: 1.1226x; 1.0182x over previous
import functools
import math

import jax
import jax.numpy as jnp
from jax.experimental import pallas as pl
from jax.experimental.pallas import tpu as pltpu

H = 16
W_BLK = 128
TOK = 256
NSUB = TOK // W_BLK
CVT_ROWS = 256
NBUF = 4

_TRANS = (((1,), (1,)), ((), ()))


def _fused_attn_kernel(x_ref, wq_hbm, wk_hbm, wv_hbm, wo_hbm, out_ref,
                       wq_s, wk_s, wv_s, wo_s, stg, s_scr, o_scr, sems,
                       *, inv_scale, d):
    i = pl.program_id(0)
    nch = d // CVT_ROWS
    srcs = (wq_hbm, wk_hbm, wv_hbm, wo_hbm)
    dsts = (wq_s, wk_s, wv_s, wo_s)
    ntot = 4 * nch

    def dma(t, buf):
        w, c = divmod(t, nch)
        return pltpu.make_async_copy(
            srcs[w].at[pl.ds(c * CVT_ROWS, CVT_ROWS), :],
            stg.at[buf], sems.at[buf])

    def proj(xv, w_s):
        return jax.lax.dot_general(xv, w_s[...], _TRANS,
                                   preferred_element_type=jnp.float32)

    def attn(q, k, v):
        for h in range(H):
            cs = slice(h * W_BLK, (h + 1) * W_BLK)
            qh = q[:, cs]
            kh = k[:, cs]
            for j in range(NSUB):
                rs = slice(j * W_BLK, (j + 1) * W_BLK)
                b = h * NSUB + j
                s_scr[b * W_BLK:(b + 1) * W_BLK, :] = jax.lax.dot_general(
                    qh[rs, :], kh[rs, :], _TRANS,
                    preferred_element_type=jnp.float32)

        sv = s_scr[...]
        sv = (sv - jnp.max(sv, axis=-1, keepdims=True)) * inv_scale
        p = jnp.exp(sv)
        p = p / jnp.sum(p, axis=-1, keepdims=True)

        for h in range(H):
            cs = slice(h * W_BLK, (h + 1) * W_BLK)
            vh = v[:, cs]
            for j in range(NSUB):
                rs = slice(j * W_BLK, (j + 1) * W_BLK)
                b = h * NSUB + j
                o_scr[rs, cs] = jnp.dot(
                    p[b * W_BLK:(b + 1) * W_BLK, :], vh[rs, :],
                    preferred_element_type=jnp.float32).astype(jnp.bfloat16)

    @pl.when(i == 0)
    def _convert_and_compute():
        state = [0]
        for pre in range(NBUF - 1):
            dma(pre, pre % NBUF).start()

        def cvt_next_weight():
            for _ in range(nch):
                t = state[0]
                buf = t % NBUF
                if t + NBUF - 1 < ntot:
                    dma(t + NBUF - 1, (t + NBUF - 1) % NBUF).start()
                dma(t, buf).wait()
                w, c = divmod(t, nch)
                dsts[w][c * CVT_ROWS:(c + 1) * CVT_ROWS, :] = (
                    stg[buf].astype(jnp.bfloat16))
                state[0] = t + 1

        xv = x_ref[...].astype(jnp.bfloat16)
        cvt_next_weight()
        q = proj(xv, wq_s)
        cvt_next_weight()
        k = proj(xv, wk_s)
        cvt_next_weight()
        v = proj(xv, wv_s)
        attn(q, k, v)
        cvt_next_weight()
        out_ref[...] = jax.lax.dot_general(
            o_scr[...], wo_s[...], _TRANS,
            preferred_element_type=jnp.float32)

    @pl.when(i != 0)
    def _compute():
        xv = x_ref[...].astype(jnp.bfloat16)
        q = proj(xv, wq_s)
        k = proj(xv, wk_s)
        v = proj(xv, wv_s)
        attn(q, k, v)
        out_ref[...] = jax.lax.dot_general(
            o_scr[...], wo_s[...], _TRANS,
            preferred_element_type=jnp.float32)


def kernel(x, Wq, Wk, Wv, Wo):
    B_, T_, D_ = x.shape
    N = B_ * T_
    Dh = D_ // H
    inv_scale = 1.0 / math.sqrt(Dh)

    x2 = x.reshape(N, D_)
    body = functools.partial(_fused_attn_kernel, inv_scale=inv_scale, d=D_)
    out = pl.pallas_call(
        body,
        grid=(N // TOK,),
        in_specs=[
            pl.BlockSpec((TOK, D_), lambda i: (i, 0)),
            pl.BlockSpec(memory_space=pl.ANY),
            pl.BlockSpec(memory_space=pl.ANY),
            pl.BlockSpec(memory_space=pl.ANY),
            pl.BlockSpec(memory_space=pl.ANY),
        ],
        out_specs=pl.BlockSpec((TOK, D_), lambda i: (i, 0)),
        out_shape=jax.ShapeDtypeStruct((N, D_), jnp.float32),
        scratch_shapes=[
            pltpu.VMEM((D_, D_), jnp.bfloat16),
            pltpu.VMEM((D_, D_), jnp.bfloat16),
            pltpu.VMEM((D_, D_), jnp.bfloat16),
            pltpu.VMEM((D_, D_), jnp.bfloat16),
            pltpu.VMEM((4, CVT_ROWS, D_), jnp.float32),
            pltpu.VMEM((H * NSUB * W_BLK, W_BLK), jnp.float32),
            pltpu.VMEM((TOK, D_), jnp.bfloat16),
            pltpu.SemaphoreType.DMA((4,)),
        ],
        compiler_params=pltpu.CompilerParams(
            dimension_semantics=("arbitrary",),
        ),
    )(x2, Wq, Wk, Wv, Wo)
    return out.reshape(B_, T_, D_)
